# baseline (speedup 1.0000x reference)
import functools

import jax
import jax.numpy as jnp
from jax import lax
from jax.experimental import pallas as pl
from jax.experimental.pallas import tpu as pltpu
from jax.experimental.pallas import tpu_sc as plsc

_N = 4096
_B = 4096
_R = 512
_D = 128
_T = _B * _R
_LANES = 65536
_CPS = _LANES // _R


def _make_sc_gather():
    info = plsc.get_sparse_core_info()
    nw = info.num_cores * info.num_subcores
    b_per_w = _B // nw
    mesh = plsc.VectorSubcoreMesh(core_axis_name="c", subcore_axis_name="s")

    @functools.partial(
        pl.kernel,
        mesh=mesh,
        out_type=jax.ShapeDtypeStruct((_B, _D), jnp.float32),
        scratch_types=[
            pltpu.VMEM((b_per_w,), jnp.int32),
            pltpu.VMEM((b_per_w, _D), jnp.float32),
            pltpu.SemaphoreType.DMA,
        ],
    )
    def sc_gather(table_hbm, idx_hbm, out_hbm, idx_v, rows_v, sem):
        wid = lax.axis_index("s") * info.num_cores + lax.axis_index("c")
        base = wid * b_per_w
        pltpu.sync_copy(idx_hbm.at[pl.ds(base, b_per_w)], idx_v)
        pltpu.async_copy(table_hbm.at[idx_v], rows_v, sem).wait()
        pltpu.sync_copy(rows_v, out_hbm.at[pl.ds(base, b_per_w)])

    return sc_gather


_sc_gather_cache = []


def _sc_gather(table, idx):
    if not _sc_gather_cache:
        _sc_gather_cache.append(_make_sc_gather())
    return _sc_gather_cache[0](table, idx)


def _prep_body(g_ref, lp_ref, t12_ref):
    g = g_ref[...]
    r = lax.broadcasted_iota(jnp.int32, (21, _D), 0)
    l = lax.broadcasted_iota(jnp.int32, (21, _D), 1)
    sel = jnp.where(r < 9, r, jnp.where(r < 18, 7 + r, 14 + r))
    w = (l == sel).astype(jnp.float32)
    t = lax.dot_general(w, g, (((1,), (1,)), ((), ())),
                        precision=lax.Precision.HIGHEST,
                        preferred_element_type=jnp.float32)
    lp = t[0:9]
    rr = lax.broadcasted_iota(jnp.int32, (9, _B), 0)
    is_z = (rr == 2) | (rr == 5) | (rr == 8)
    lp_ref[...] = jnp.where(is_z, -jnp.abs(lp), lp)
    t12_ref[...] = jnp.abs(t[9:21])


def _prep(gathered):
    return pl.pallas_call(
        _prep_body,
        out_shape=[
            jax.ShapeDtypeStruct((9, _B), jnp.float32),
            jax.ShapeDtypeStruct((12, _B), jnp.float32),
        ],
    )(gathered)


def _bcast_body(lp_t_ref, t12_ref, lp_ref):
    t = lp_t_ref[0:8, :]
    for k in range(_LANES // _B):
        lp_ref[:, k * _B:(k + 1) * _B] = t


def _bcast(lp_t, t12):
    return pl.pallas_call(
        _bcast_body,
        grid=(_T // _LANES,),
        in_specs=[
            pl.BlockSpec((9, _B), lambda i: (0, 0)),
            pl.BlockSpec((12, _CPS), lambda i: (0, i)),
        ],
        out_specs=[
            pl.BlockSpec((8, _LANES), lambda i: (0, i)),
        ],
        out_shape=[
            jax.ShapeDtypeStruct((8, _T), jnp.float32),
        ],
    )(lp_t, t12)


def kernel(light1_pos_xy, light1_pos_z, light1_intensity,
           light2_pos_xy, light2_pos_z, light2_intensity,
           light3_pos_xy, light3_pos_z, light3_intensity,
           beta, idx):
    pad7 = jnp.zeros((_N, 7), jnp.float32)
    pad93 = jnp.zeros((_N, _D - 35), jnp.float32)
    table = jnp.concatenate(
        [light1_pos_xy, light1_pos_z, light2_pos_xy, light2_pos_z,
         light3_pos_xy, light3_pos_z, pad7,
         light1_intensity, light2_intensity, light3_intensity, pad7,
         beta, pad93],
        axis=1)
    gathered = _sc_gather(table, idx.astype(jnp.int32))
    lp_t, t12 = _prep(gathered)
    (lp2d,) = _bcast(lp_t, t12)
    return (lp2d.T, lp2d.T, lp2d.T)


# speedup vs baseline: 1.1093x; 1.1093x over previous
import functools

import jax
import jax.numpy as jnp
from jax import lax
from jax.experimental import pallas as pl
from jax.experimental.pallas import tpu as pltpu
from jax.experimental.pallas import tpu_sc as plsc

_N = 4096
_B = 4096
_R = 512
_D = 128
_T = _B * _R
_CH = 65536
_NCH = _T // _CH
_CPS = _CH // _R
_NS = 4


def _make_sc_gather():
    info = plsc.get_sparse_core_info()
    nw = info.num_cores * info.num_subcores
    b_per_w = _B // nw
    mesh = plsc.VectorSubcoreMesh(core_axis_name="c", subcore_axis_name="s")

    @functools.partial(
        pl.kernel,
        mesh=mesh,
        out_type=jax.ShapeDtypeStruct((_B, _D), jnp.float32),
        scratch_types=[
            pltpu.VMEM((b_per_w,), jnp.int32),
            pltpu.VMEM((b_per_w, _D), jnp.float32),
            pltpu.SemaphoreType.DMA,
        ],
    )
    def sc_gather(table_hbm, idx_hbm, out_hbm, idx_v, rows_v, sem):
        wid = lax.axis_index("s") * info.num_cores + lax.axis_index("c")
        base = wid * b_per_w
        pltpu.sync_copy(idx_hbm.at[pl.ds(base, b_per_w)], idx_v)
        pltpu.async_copy(table_hbm.at[idx_v], rows_v, sem).wait()
        pltpu.sync_copy(rows_v, out_hbm.at[pl.ds(base, b_per_w)])

    return sc_gather


_sc_gather_cache = []


def _sc_gather(table, idx):
    if not _sc_gather_cache:
        _sc_gather_cache.append(_make_sc_gather())
    return _sc_gather_cache[0](table, idx)


def _prep_body(g_ref, lp_ref, t12_ref):
    g = g_ref[...]
    r = lax.broadcasted_iota(jnp.int32, (21, _D), 0)
    l = lax.broadcasted_iota(jnp.int32, (21, _D), 1)
    sel = jnp.where(r < 9, r, jnp.where(r < 18, 7 + r, 14 + r))
    w = (l == sel).astype(jnp.float32)
    t = lax.dot_general(w, g, (((1,), (1,)), ((), ())),
                        precision=lax.Precision.HIGHEST,
                        preferred_element_type=jnp.float32)
    lp = t[0:9]
    rr = lax.broadcasted_iota(jnp.int32, (9, _B), 0)
    is_z = (rr == 2) | (rr == 5) | (rr == 8)
    lp_ref[...] = jnp.where(is_z, -jnp.abs(lp), lp)
    t12_ref[...] = jnp.abs(t[9:21])


def _prep(gathered):
    return pl.pallas_call(
        _prep_body,
        out_shape=[
            jax.ShapeDtypeStruct((9, _B), jnp.float32),
            jax.ShapeDtypeStruct((12, _B), jnp.float32),
        ],
    )(gathered)


def _cp(buf, hbm, j, sem):
    return pltpu.make_async_copy(buf, hbm.at[:, pl.ds(j * _CH, _CH)], sem)


def _bcast_body(lp_t_ref, t12_ref, lp_hbm, li_hbm, bt_hbm,
                lp_buf, li_buf, bt_buf, lp_sem, li_sem, bt_sem):
    j = pl.program_id(0)

    @pl.when(j == 0)
    def _():
        t = lp_t_ref[...]
        for k in range(_CH // _B):
            lp_buf[:, k * _B:(k + 1) * _B] = t

    s = lax.rem(j, _NS)
    jw = j - _NS

    for ss in range(_NS):
        @pl.when(s == ss)
        def _(ss=ss):
            @pl.when(j >= _NS)
            def _():
                _cp(lp_buf, lp_hbm, jw, lp_sem.at[ss]).wait()
                _cp(li_buf.at[ss], li_hbm, jw, li_sem.at[ss]).wait()
                _cp(bt_buf.at[ss], bt_hbm, jw, bt_sem.at[ss]).wait()

            sv = t12_ref[:, pl.ds(j * _CPS, _CPS)]
            for k in range(_CPS):
                li_buf[ss, :, k * _R:(k + 1) * _R] = jnp.broadcast_to(
                    sv[0:9, k:k + 1], (9, _R))
                bt_buf[ss, :, k * _R:(k + 1) * _R] = jnp.broadcast_to(
                    sv[9:12, k:k + 1], (3, _R))
            _cp(lp_buf, lp_hbm, j, lp_sem.at[ss]).start()
            _cp(li_buf.at[ss], li_hbm, j, li_sem.at[ss]).start()
            _cp(bt_buf.at[ss], bt_hbm, j, bt_sem.at[ss]).start()

    @pl.when(j == _NCH - 1)
    def _():
        for jd in range(_NCH - _NS, _NCH):
            sd = jd % _NS
            _cp(lp_buf, lp_hbm, jd, lp_sem.at[sd]).wait()
            _cp(li_buf.at[sd], li_hbm, jd, li_sem.at[sd]).wait()
            _cp(bt_buf.at[sd], bt_hbm, jd, bt_sem.at[sd]).wait()


def _bcast(lp_t, t12):
    return pl.pallas_call(
        _bcast_body,
        grid=(_NCH,),
        in_specs=[
            pl.BlockSpec((9, _B), lambda i: (0, 0)),
            pl.BlockSpec((12, _B), lambda i: (0, 0)),
        ],
        out_specs=[
            pl.BlockSpec(memory_space=pl.ANY),
            pl.BlockSpec(memory_space=pl.ANY),
            pl.BlockSpec(memory_space=pl.ANY),
        ],
        out_shape=[
            jax.ShapeDtypeStruct((9, _T), jnp.float32),
            jax.ShapeDtypeStruct((9, _T), jnp.float32),
            jax.ShapeDtypeStruct((3, _T), jnp.float32),
        ],
        scratch_shapes=[
            pltpu.VMEM((9, _CH), jnp.float32),
            pltpu.VMEM((_NS, 9, _CH), jnp.float32),
            pltpu.VMEM((_NS, 3, _CH), jnp.float32),
            pltpu.SemaphoreType.DMA((_NS,)),
            pltpu.SemaphoreType.DMA((_NS,)),
            pltpu.SemaphoreType.DMA((_NS,)),
        ],
    )(lp_t, t12)


def kernel(light1_pos_xy, light1_pos_z, light1_intensity,
           light2_pos_xy, light2_pos_z, light2_intensity,
           light3_pos_xy, light3_pos_z, light3_intensity,
           beta, idx):
    pad7 = jnp.zeros((_N, 7), jnp.float32)
    pad93 = jnp.zeros((_N, _D - 35), jnp.float32)
    table = jnp.concatenate(
        [light1_pos_xy, light1_pos_z, light2_pos_xy, light2_pos_z,
         light3_pos_xy, light3_pos_z, pad7,
         light1_intensity, light2_intensity, light3_intensity, pad7,
         beta, pad93],
        axis=1)
    gathered = _sc_gather(table, idx.astype(jnp.int32))
    lp_t, t12 = _prep(gathered)
    lp2d, li2d, bt2d = _bcast(lp_t, t12)
    return (lp2d.T, li2d.T, bt2d.T)


# speedup vs baseline: 1.1101x; 1.0008x over previous
import functools

import jax
import jax.numpy as jnp
from jax import lax
from jax.experimental import pallas as pl
from jax.experimental.pallas import tpu as pltpu
from jax.experimental.pallas import tpu_sc as plsc

_N = 4096
_B = 4096
_R = 512
_D = 128
_T = _B * _R
_CH = 65536
_NCH = _T // _CH
_CPS = _CH // _R
_NS = 4


def _make_sc_gather():
    info = plsc.get_sparse_core_info()
    nw = info.num_cores * info.num_subcores
    b_per_w = _B // nw
    mesh = plsc.VectorSubcoreMesh(core_axis_name="c", subcore_axis_name="s")

    @functools.partial(
        pl.kernel,
        mesh=mesh,
        out_type=jax.ShapeDtypeStruct((_B, _D), jnp.float32),
        scratch_types=[
            pltpu.VMEM((b_per_w,), jnp.int32),
            pltpu.VMEM((b_per_w, _D), jnp.float32),
            pltpu.SemaphoreType.DMA,
        ],
    )
    def sc_gather(table_hbm, idx_hbm, out_hbm, idx_v, rows_v, sem):
        wid = lax.axis_index("s") * info.num_cores + lax.axis_index("c")
        base = wid * b_per_w
        pltpu.sync_copy(idx_hbm.at[pl.ds(base, b_per_w)], idx_v)
        pltpu.async_copy(table_hbm.at[idx_v], rows_v, sem).wait()
        pltpu.sync_copy(rows_v, out_hbm.at[pl.ds(base, b_per_w)])

    return sc_gather


_sc_gather_cache = []


def _sc_gather(table, idx):
    if not _sc_gather_cache:
        _sc_gather_cache.append(_make_sc_gather())
    return _sc_gather_cache[0](table, idx)


def _prep_body(g_ref, lp_ref, t12_ref):
    g = g_ref[...]
    r = lax.broadcasted_iota(jnp.int32, (21, _D), 0)
    l = lax.broadcasted_iota(jnp.int32, (21, _D), 1)
    sel = jnp.where(r < 9, r, jnp.where(r < 18, 7 + r, 14 + r))
    w = (l == sel).astype(jnp.float32)
    t = lax.dot_general(w, g, (((1,), (1,)), ((), ())),
                        precision=lax.Precision.HIGHEST,
                        preferred_element_type=jnp.float32)
    lp = t[0:9]
    rr = lax.broadcasted_iota(jnp.int32, (9, _B), 0)
    is_z = (rr == 2) | (rr == 5) | (rr == 8)
    lp_ref[...] = jnp.where(is_z, -jnp.abs(lp), lp)
    t12_ref[...] = jnp.abs(t[9:21])


def _prep(gathered):
    return pl.pallas_call(
        _prep_body,
        out_shape=[
            jax.ShapeDtypeStruct((9, _B), jnp.float32),
            jax.ShapeDtypeStruct((12, _B), jnp.float32),
        ],
    )(gathered)


def _cpA(buf, hbm, j, sem):
    return pltpu.make_async_copy(
        buf.at[pl.ds(0, 8), :], hbm.at[pl.ds(0, 8), pl.ds(j * _CH, _CH)], sem)


def _cp8(buf, hbm, j, sem):
    return pltpu.make_async_copy(
        buf.at[pl.ds(8, 1), :], hbm.at[pl.ds(8, 1), pl.ds(j * _CH, _CH)], sem)


def _cpB(buf, hbm, j, sem):
    return pltpu.make_async_copy(buf, hbm.at[:, pl.ds(j * _CH, _CH)], sem)


def _bcast_body(lp_t_ref, t12_ref, lp_hbm, li_hbm, bt_hbm,
                lp_buf, li_buf, bt_buf,
                lp_sem, li_sem, bt_sem, lp8_sem, li8_sem):
    j = pl.program_id(0)

    @pl.when(j == 0)
    def _():
        t = lp_t_ref[...]
        for k in range(_CH // _B):
            lp_buf[:, k * _B:(k + 1) * _B] = t

    s = lax.rem(j, _NS)
    jw = j - _NS

    for ss in range(_NS):
        @pl.when(s == ss)
        def _(ss=ss):
            @pl.when(j >= _NS)
            def _():
                _cpA(lp_buf, lp_hbm, jw, lp_sem.at[ss]).wait()
                _cp8(lp_buf, lp_hbm, jw, lp8_sem.at[ss]).wait()
                _cpA(li_buf.at[ss], li_hbm, jw, li_sem.at[ss]).wait()
                _cp8(li_buf.at[ss], li_hbm, jw, li8_sem.at[ss]).wait()
                _cpB(bt_buf.at[ss], bt_hbm, jw, bt_sem.at[ss]).wait()

            sv = t12_ref[:, pl.ds(j * _CPS, _CPS)]
            for k in range(_CPS):
                li_buf[ss, :, k * _R:(k + 1) * _R] = jnp.broadcast_to(
                    sv[0:9, k:k + 1], (9, _R))
                bt_buf[ss, :, k * _R:(k + 1) * _R] = jnp.broadcast_to(
                    sv[9:12, k:k + 1], (3, _R))
            _cp8(lp_buf, lp_hbm, j, lp8_sem.at[ss]).start()
            _cp8(li_buf.at[ss], li_hbm, j, li8_sem.at[ss]).start()
            _cpB(bt_buf.at[ss], bt_hbm, j, bt_sem.at[ss]).start()
            _cpA(lp_buf, lp_hbm, j, lp_sem.at[ss]).start()
            _cpA(li_buf.at[ss], li_hbm, j, li_sem.at[ss]).start()

    @pl.when(j == _NCH - 1)
    def _():
        for jd in range(_NCH - _NS, _NCH):
            sd = jd % _NS
            _cpA(lp_buf, lp_hbm, jd, lp_sem.at[sd]).wait()
            _cp8(lp_buf, lp_hbm, jd, lp8_sem.at[sd]).wait()
            _cpA(li_buf.at[sd], li_hbm, jd, li_sem.at[sd]).wait()
            _cp8(li_buf.at[sd], li_hbm, jd, li8_sem.at[sd]).wait()
            _cpB(bt_buf.at[sd], bt_hbm, jd, bt_sem.at[sd]).wait()


def _bcast(lp_t, t12):
    return pl.pallas_call(
        _bcast_body,
        grid=(_NCH,),
        in_specs=[
            pl.BlockSpec((9, _B), lambda i: (0, 0)),
            pl.BlockSpec((12, _B), lambda i: (0, 0)),
        ],
        out_specs=[
            pl.BlockSpec(memory_space=pl.ANY),
            pl.BlockSpec(memory_space=pl.ANY),
            pl.BlockSpec(memory_space=pl.ANY),
        ],
        out_shape=[
            jax.ShapeDtypeStruct((9, _T), jnp.float32),
            jax.ShapeDtypeStruct((9, _T), jnp.float32),
            jax.ShapeDtypeStruct((3, _T), jnp.float32),
        ],
        scratch_shapes=[
            pltpu.VMEM((9, _CH), jnp.float32),
            pltpu.VMEM((_NS, 9, _CH), jnp.float32),
            pltpu.VMEM((_NS, 3, _CH), jnp.float32),
            pltpu.SemaphoreType.DMA((_NS,)),
            pltpu.SemaphoreType.DMA((_NS,)),
            pltpu.SemaphoreType.DMA((_NS,)),
            pltpu.SemaphoreType.DMA((_NS,)),
            pltpu.SemaphoreType.DMA((_NS,)),
        ],
    )(lp_t, t12)


def kernel(light1_pos_xy, light1_pos_z, light1_intensity,
           light2_pos_xy, light2_pos_z, light2_intensity,
           light3_pos_xy, light3_pos_z, light3_intensity,
           beta, idx):
    pad7 = jnp.zeros((_N, 7), jnp.float32)
    pad93 = jnp.zeros((_N, _D - 35), jnp.float32)
    table = jnp.concatenate(
        [light1_pos_xy, light1_pos_z, light2_pos_xy, light2_pos_z,
         light3_pos_xy, light3_pos_z, pad7,
         light1_intensity, light2_intensity, light3_intensity, pad7,
         beta, pad93],
        axis=1)
    gathered = _sc_gather(table, idx.astype(jnp.int32))
    lp_t, t12 = _prep(gathered)
    lp2d, li2d, bt2d = _bcast(lp_t, t12)
    return (lp2d.T, li2d.T, bt2d.T)


# speedup vs baseline: 1.1183x; 1.0074x over previous
import functools

import jax
import jax.numpy as jnp
from jax import lax
from jax.experimental import pallas as pl
from jax.experimental.pallas import tpu as pltpu
from jax.experimental.pallas import tpu_sc as plsc

_N = 4096
_B = 4096
_R = 512
_D = 128
_T = _B * _R
_LANES = 131072
_CPS = _LANES // _R


def _make_sc_gather():
    info = plsc.get_sparse_core_info()
    nw = info.num_cores * info.num_subcores
    b_per_w = _B // nw
    mesh = plsc.VectorSubcoreMesh(core_axis_name="c", subcore_axis_name="s")

    @functools.partial(
        pl.kernel,
        mesh=mesh,
        out_type=jax.ShapeDtypeStruct((_B, _D), jnp.float32),
        scratch_types=[
            pltpu.VMEM((b_per_w,), jnp.int32),
            pltpu.VMEM((b_per_w, _D), jnp.float32),
            pltpu.SemaphoreType.DMA,
        ],
    )
    def sc_gather(table_hbm, idx_hbm, out_hbm, idx_v, rows_v, sem):
        wid = lax.axis_index("s") * info.num_cores + lax.axis_index("c")
        base = wid * b_per_w
        pltpu.sync_copy(idx_hbm.at[pl.ds(base, b_per_w)], idx_v)
        pltpu.async_copy(table_hbm.at[idx_v], rows_v, sem).wait()
        pltpu.sync_copy(rows_v, out_hbm.at[pl.ds(base, b_per_w)])

    return sc_gather


_sc_gather_cache = []


def _sc_gather(table, idx):
    if not _sc_gather_cache:
        _sc_gather_cache.append(_make_sc_gather())
    return _sc_gather_cache[0](table, idx)


def _prep_body(g_ref, lp_ref, t12_ref):
    g = g_ref[...]
    r = lax.broadcasted_iota(jnp.int32, (21, _D), 0)
    l = lax.broadcasted_iota(jnp.int32, (21, _D), 1)
    sel = jnp.where(r < 9, r, jnp.where(r < 18, 7 + r, 14 + r))
    w = (l == sel).astype(jnp.float32)
    t = lax.dot_general(w, g, (((1,), (1,)), ((), ())),
                        precision=lax.Precision.HIGHEST,
                        preferred_element_type=jnp.float32)
    lp = t[0:9]
    rr = lax.broadcasted_iota(jnp.int32, (9, _B), 0)
    is_z = (rr == 2) | (rr == 5) | (rr == 8)
    lp_ref[...] = jnp.where(is_z, -jnp.abs(lp), lp)
    t12_ref[...] = jnp.abs(t[9:21])


def _prep(gathered):
    return pl.pallas_call(
        _prep_body,
        out_shape=[
            jax.ShapeDtypeStruct((9, _B), jnp.float32),
            jax.ShapeDtypeStruct((12, _B), jnp.float32),
        ],
    )(gathered)


def _bcast_body(lp_t_ref, t12_ref, lp_ref, li_ref, bt_ref):
    t = lp_t_ref[...]
    for k in range(_LANES // _B):
        lp_ref[:, k * _B:(k + 1) * _B] = t
    s = t12_ref[...]
    for k in range(_CPS):
        li_ref[:, k * _R:(k + 1) * _R] = jnp.broadcast_to(
            s[0:9, k:k + 1], (9, _R))
        bt_ref[:, k * _R:(k + 1) * _R] = jnp.broadcast_to(
            s[9:12, k:k + 1], (3, _R))


def _bcast(lp_t, t12):
    return pl.pallas_call(
        _bcast_body,
        grid=(_T // _LANES,),
        in_specs=[
            pl.BlockSpec((9, _B), lambda i: (0, 0)),
            pl.BlockSpec((12, _CPS), lambda i: (0, i)),
        ],
        out_specs=[
            pl.BlockSpec((9, _LANES), lambda i: (0, i)),
            pl.BlockSpec((9, _LANES), lambda i: (0, i)),
            pl.BlockSpec((3, _LANES), lambda i: (0, i)),
        ],
        out_shape=[
            jax.ShapeDtypeStruct((9, _T), jnp.float32),
            jax.ShapeDtypeStruct((9, _T), jnp.float32),
            jax.ShapeDtypeStruct((3, _T), jnp.float32),
        ],
    )(lp_t, t12)


def kernel(light1_pos_xy, light1_pos_z, light1_intensity,
           light2_pos_xy, light2_pos_z, light2_intensity,
           light3_pos_xy, light3_pos_z, light3_intensity,
           beta, idx):
    pad7 = jnp.zeros((_N, 7), jnp.float32)
    pad93 = jnp.zeros((_N, _D - 35), jnp.float32)
    table = jnp.concatenate(
        [light1_pos_xy, light1_pos_z, light2_pos_xy, light2_pos_z,
         light3_pos_xy, light3_pos_z, pad7,
         light1_intensity, light2_intensity, light3_intensity, pad7,
         beta, pad93],
        axis=1)
    gathered = _sc_gather(table, idx.astype(jnp.int32))
    lp_t, t12 = _prep(gathered)
    lp2d, li2d, bt2d = _bcast(lp_t, t12)
    return (lp2d.T, li2d.T, bt2d.T)
